# baseline (speedup 1.0000x reference)
import functools

import jax
import jax.numpy as jnp
from jax import lax
from jax.experimental import pallas as pl
from jax.experimental.pallas import tpu as pltpu
from jax.experimental.pallas import tpu_sc as plsc


def _tc_fused_body(srow_ref, x_ref, a_ref, b_ref, o_ref):
    h = jnp.dot(x_ref[...], a_ref[...], preferred_element_type=jnp.float32)
    h = h * srow_ref[...]
    o_ref[...] = jnp.dot(h, b_ref[...], preferred_element_type=jnp.float32)


def _sc_gather_scale_rows(scale_table, slot_ids, T, ER):
    info = plsc.get_sparse_core_info()
    NC, NS = info.num_cores, info.num_subcores
    NW = NC * NS
    CH = 128
    per_w = T // NW
    n_chunks = per_w // CH
    mesh = plsc.VectorSubcoreMesh(core_axis_name="c", subcore_axis_name="s")

    @functools.partial(
        pl.kernel, mesh=mesh,
        out_type=jax.ShapeDtypeStruct((T, ER), jnp.float32),
        scratch_types=[
            pltpu.VMEM((CH,), jnp.int32),
            pltpu.VMEM((CH, ER), jnp.float32),
            pltpu.SemaphoreType.DMA,
        ],
    )
    def sc_gather(table_hbm, idx_hbm, out_hbm, idx_v, rows_v, sem):
        wid = lax.axis_index("s") * NC + lax.axis_index("c")
        base = wid * per_w
        for j in range(n_chunks):
            off = base + j * CH
            pltpu.sync_copy(idx_hbm.at[pl.ds(off, CH)], idx_v)
            pltpu.async_copy(table_hbm.at[idx_v], rows_v, sem).wait()
            pltpu.sync_copy(rows_v, out_hbm.at[pl.ds(off, CH)])

    return sc_gather(scale_table, slot_ids)


def kernel(x, token_to_slot, lora_a, lora_b, lora_scaling):
    T, D = x.shape
    E, _, R = lora_a.shape
    D_out = lora_b.shape[-1]
    ER = E * R

    a_cat = jnp.transpose(lora_a, (1, 0, 2)).reshape(D, ER)
    b_cat = lora_b.reshape(ER, D_out)
    block = (jnp.arange(ER)[None, :] // R) == jnp.arange(E)[:, None]
    scale_table = jnp.where(block, lora_scaling[:, None], 0.0)

    s_rows = _sc_gather_scale_rows(scale_table, token_to_slot.astype(jnp.int32),
                                   T, ER)

    BT = 1536
    grid = (pl.cdiv(T, BT),)

    return pl.pallas_call(
        _tc_fused_body,
        grid=grid,
        in_specs=[
            pl.BlockSpec((BT, ER), lambda i: (i, 0)),
            pl.BlockSpec((BT, D), lambda i: (i, 0)),
            pl.BlockSpec((D, ER), lambda i: (0, 0)),
            pl.BlockSpec((ER, D_out), lambda i: (0, 0)),
        ],
        out_specs=pl.BlockSpec((BT, D_out), lambda i: (i, 0)),
        out_shape=jax.ShapeDtypeStruct((T, D_out), x.dtype),
        compiler_params=pltpu.CompilerParams(
            dimension_semantics=("parallel",),
        ),
    )(s_rows, x, a_cat, b_cat)


# speedup vs baseline: 2.0310x; 2.0310x over previous
import functools

import jax
import jax.numpy as jnp
from jax import lax
from jax.experimental import pallas as pl
from jax.experimental.pallas import tpu as pltpu


def _fused_lora_body(slot_ref, scale_ref, x_ref, a_ref, b_ref, o_ref, *, rank):
    h = jnp.dot(x_ref[...], a_ref[...], preferred_element_type=jnp.float32)
    slots = slot_ref[...]
    er = h.shape[1]
    col_slot = lax.broadcasted_iota(jnp.int32, (h.shape[0], er), 1) // rank
    h = jnp.where(col_slot == slots, h * scale_ref[...], 0.0)
    o_ref[...] = jnp.dot(h, b_ref[...], preferred_element_type=jnp.float32)


def kernel(x, token_to_slot, lora_a, lora_b, lora_scaling):
    T, D = x.shape
    E, _, R = lora_a.shape
    D_out = lora_b.shape[-1]
    ER = E * R

    a_cat = jnp.transpose(lora_a, (1, 0, 2)).reshape(D, ER)
    b_cat = lora_b.reshape(ER, D_out)
    scale_vec = jnp.repeat(lora_scaling, R).reshape(1, ER)
    slots2 = token_to_slot.reshape(T, 1).astype(jnp.int32)

    BT = 1664
    grid = (pl.cdiv(T, BT),)

    return pl.pallas_call(
        functools.partial(_fused_lora_body, rank=R),
        grid=grid,
        in_specs=[
            pl.BlockSpec((BT, 1), lambda i: (i, 0)),
            pl.BlockSpec((1, ER), lambda i: (0, 0)),
            pl.BlockSpec((BT, D), lambda i: (i, 0)),
            pl.BlockSpec((D, ER), lambda i: (0, 0)),
            pl.BlockSpec((ER, D_out), lambda i: (0, 0)),
        ],
        out_specs=pl.BlockSpec((BT, D_out), lambda i: (i, 0)),
        out_shape=jax.ShapeDtypeStruct((T, D_out), x.dtype),
        compiler_params=pltpu.CompilerParams(
            dimension_semantics=("parallel",),
        ),
    )(slots2, scale_vec, x, a_cat, b_cat)
